# baseline (speedup 1.0000x reference)
import functools

import jax
import jax.numpy as jnp
from jax import lax
from jax.experimental import pallas as pl
from jax.experimental.pallas import tpu as pltpu
from jax.experimental.pallas import tpu_sc as plsc

H = 128
RB = 512

NG = 100000
NM = 10000
NG_PAD = 100352
NM_PAD = 10240
NM_ACC = 10368
WIN = 12544
NWIN = 8
ACC_G = 12672
GARB_FAR = 1 << 20


def _ln(y, g, beta):
    mu = jnp.mean(y, axis=-1, keepdims=True)
    var = jnp.mean((y - mu) ** 2, axis=-1, keepdims=True)
    return (y - mu) * lax.rsqrt(var + 1e-5) * g + beta


def _silu(x):
    return x * jax.nn.sigmoid(x)


def _dot(a, b):
    return jnp.dot(a, b, preferred_element_type=jnp.float32)


def _mlp_body(x_ref, w1_ref, b1_ref, w2_ref, b2_ref, g_ref, beta_ref, o_ref,
              *, norm, res):
    x = x_ref[...]
    h = _silu(_dot(x, w1_ref[...]) + b1_ref[...])
    y = _dot(h, w2_ref[...]) + b2_ref[...]
    if norm:
        y = _ln(y, g_ref[...], beta_ref[...])
    if res:
        y = y + x
    o_ref[...] = y


def _mlp_rows(x, p, *, norm=True, res=False, w1=None):
    n, din = x.shape
    w1 = p["W1"] if w1 is None else w1
    dh = w1.shape[1]
    w2 = p["W2"]
    dout = w2.shape[1]
    if norm:
        g, beta = p["g"].reshape(1, -1), p["beta"].reshape(1, -1)
    else:
        g = jnp.ones((1, dout), jnp.float32)
        beta = jnp.zeros((1, dout), jnp.float32)
    return pl.pallas_call(
        functools.partial(_mlp_body, norm=norm, res=res),
        grid=(n // RB,),
        in_specs=[
            pl.BlockSpec((RB, din), lambda i: (i, 0)),
            pl.BlockSpec((din, dh), lambda i: (0, 0)),
            pl.BlockSpec((1, dh), lambda i: (0, 0)),
            pl.BlockSpec((dh, dout), lambda i: (0, 0)),
            pl.BlockSpec((1, dout), lambda i: (0, 0)),
            pl.BlockSpec((1, dout), lambda i: (0, 0)),
            pl.BlockSpec((1, dout), lambda i: (0, 0)),
        ],
        out_specs=pl.BlockSpec((RB, dout), lambda i: (i, 0)),
        out_shape=jax.ShapeDtypeStruct((n, dout), jnp.float32),
    )(x, w1, p["b1"].reshape(1, -1), w2, p["b2"].reshape(1, -1), g, beta)


def _edge_body(e_ref, ga_ref, gb_ref, w1_ref, b1_ref, w2_ref, b2_ref,
               g_ref, beta_ref, o_ref):
    e = e_ref[...]
    h = _silu(_dot(e, w1_ref[...]) + ga_ref[...] + gb_ref[...] + b1_ref[...])
    y = _dot(h, w2_ref[...]) + b2_ref[...]
    o_ref[...] = e + _ln(y, g_ref[...], beta_ref[...])


def _edge_update(e, ga, gb, w1e, p):
    n = e.shape[0]
    return pl.pallas_call(
        _edge_body,
        grid=(n // RB,),
        in_specs=[
            pl.BlockSpec((RB, H), lambda i: (i, 0)),
            pl.BlockSpec((RB, H), lambda i: (i, 0)),
            pl.BlockSpec((RB, H), lambda i: (i, 0)),
            pl.BlockSpec((H, H), lambda i: (0, 0)),
            pl.BlockSpec((1, H), lambda i: (0, 0)),
            pl.BlockSpec((H, H), lambda i: (0, 0)),
            pl.BlockSpec((1, H), lambda i: (0, 0)),
            pl.BlockSpec((1, H), lambda i: (0, 0)),
            pl.BlockSpec((1, H), lambda i: (0, 0)),
        ],
        out_specs=pl.BlockSpec((RB, H), lambda i: (i, 0)),
        out_shape=jax.ShapeDtypeStruct((n, H), jnp.float32),
    )(e, ga, gb, w1e, p["b1"].reshape(1, -1), p["W2"],
      p["b2"].reshape(1, -1), p["g"].reshape(1, -1), p["beta"].reshape(1, -1))


def _node_body(x_ref, a1_ref, a2_ref, w1x_ref, w1a_ref, b1_ref, w2_ref,
               b2_ref, g_ref, beta_ref, o_ref, *, two_agg):
    x = x_ref[...]
    agg = a1_ref[...]
    if two_agg:
        agg = agg + a2_ref[...]
    h = _silu(_dot(x, w1x_ref[...]) + _dot(agg, w1a_ref[...]) + b1_ref[...])
    y = _dot(h, w2_ref[...]) + b2_ref[...]
    o_ref[...] = x + _ln(y, g_ref[...], beta_ref[...])


def _node_update(x, a1, a2, p):
    n = x.shape[0]
    two_agg = a2 is not None
    if not two_agg:
        a2 = a1
    w1 = p["W1"]
    return pl.pallas_call(
        functools.partial(_node_body, two_agg=two_agg),
        grid=(n // RB,),
        in_specs=[pl.BlockSpec((RB, H), lambda i: (i, 0))] * 3
        + [
            pl.BlockSpec((H, H), lambda i: (0, 0)),
            pl.BlockSpec((H, H), lambda i: (0, 0)),
            pl.BlockSpec((1, H), lambda i: (0, 0)),
            pl.BlockSpec((H, H), lambda i: (0, 0)),
            pl.BlockSpec((1, H), lambda i: (0, 0)),
            pl.BlockSpec((1, H), lambda i: (0, 0)),
            pl.BlockSpec((1, H), lambda i: (0, 0)),
        ],
        out_specs=pl.BlockSpec((RB, H), lambda i: (i, 0)),
        out_shape=jax.ShapeDtypeStruct((n, H), jnp.float32),
    )(x, a1, a2, w1[:H], w1[H:], p["b1"].reshape(1, -1), p["W2"],
      p["b2"].reshape(1, -1), p["g"].reshape(1, -1), p["beta"].reshape(1, -1))


def _mm_body(x_ref, w_ref, o_ref):
    o_ref[...] = _dot(x_ref[...], w_ref[...])


def _mm(x, w):
    n = x.shape[0]
    dout = w.shape[1]
    return pl.pallas_call(
        _mm_body,
        grid=(n // RB,),
        in_specs=[
            pl.BlockSpec((RB, H), lambda i: (i, 0)),
            pl.BlockSpec((H, dout), lambda i: (0, 0)),
        ],
        out_specs=pl.BlockSpec((RB, dout), lambda i: (i, 0)),
        out_shape=jax.ShapeDtypeStruct((n, dout), jnp.float32),
    )(x, w)


def _sc_mesh():
    return plsc.VectorSubcoreMesh(core_axis_name="c", subcore_axis_name="s")


def _drain(src, dst, sem):
    pltpu.make_async_copy(src, dst, sem).wait()


def _sc_gather2(table_a, table_b, idx_a, idx_b):
    (e,) = idx_a.shape
    per_w = e // 32
    CH = 64
    nch = per_w // CH

    @functools.partial(
        pl.kernel,
        out_type=(jax.ShapeDtypeStruct((e, H), jnp.float32),
                  jax.ShapeDtypeStruct((e, H), jnp.float32)),
        mesh=_sc_mesh(),
        scratch_types=[
            pltpu.VMEM((2, 4, CH), jnp.int32),
            pltpu.VMEM((2, 4, CH, H), jnp.float32),
            pltpu.SemaphoreType.DMA((2, 4)),
            pltpu.SemaphoreType.DMA((2, 4)),
            pltpu.SemaphoreType.DMA((2, 4)),
        ],
    )
    def k(ta, tb, ia_h, ib_h, oa, ob, iv, rv, si, sg, sw):
        c = lax.axis_index("c")
        s = lax.axis_index("s")
        base = (s * 2 + c) * per_w

        def idx_load(i, isl):
            off = base + i * CH
            pltpu.async_copy(ia_h.at[pl.ds(off, CH)], iv.at[0, isl], si.at[0, isl])
            pltpu.async_copy(ib_h.at[pl.ds(off, CH)], iv.at[1, isl], si.at[1, isl])

        def wait_idx(isl):
            _drain(ia_h.at[pl.ds(base, CH)], iv.at[0, isl], si.at[0, isl])
            _drain(ib_h.at[pl.ds(base, CH)], iv.at[1, isl], si.at[1, isl])

        def wait_gather(rs):
            _drain(ta.at[iv.at[0, 0]], rv.at[0, rs], sg.at[0, rs])
            _drain(tb.at[iv.at[1, 0]], rv.at[1, rs], sg.at[1, rs])

        def start_wb(i, rs):
            off = base + i * CH
            pltpu.async_copy(rv.at[0, rs], oa.at[pl.ds(off, CH)], sw.at[0, rs])
            pltpu.async_copy(rv.at[1, rs], ob.at[pl.ds(off, CH)], sw.at[1, rs])

        def wait_wb(rs):
            _drain(rv.at[0, rs], oa.at[pl.ds(base, CH)], sw.at[0, rs])
            _drain(rv.at[1, rs], ob.at[pl.ds(base, CH)], sw.at[1, rs])

        idx_load(0, 0)
        idx_load(1, 1)

        @pl.loop(0, nch // 4)
        def _(j):
            for k4 in range(4):
                i = j * 4 + k4

                @pl.when(i >= 4)
                def _():
                    wait_wb(k4)

                wait_idx(k4)
                pltpu.async_copy(ta.at[iv.at[0, k4]], rv.at[0, k4], sg.at[0, k4])
                pltpu.async_copy(tb.at[iv.at[1, k4]], rv.at[1, k4], sg.at[1, k4])

                @pl.when(i + 2 < nch)
                def _():
                    idx_load(i + 2, (k4 + 2) % 4)

                @pl.when(i >= 3)
                def _():
                    wait_gather((k4 + 1) % 4)
                    start_wb(i - 3, (k4 + 1) % 4)

        for k4, i in ((1, nch - 3), (2, nch - 2), (3, nch - 1)):
            wait_gather(k4)
            start_wb(i, k4)
        for k4 in range(4):
            wait_wb(k4)

    return k(table_a, table_b, idx_a, idx_b)


def _sc_scatter_mesh(enew, dst, zeros_rows):
    e = dst.shape[0]
    per_t = e // 32
    CH = 64
    nch = per_t // CH

    @functools.partial(
        pl.kernel,
        out_type=jax.ShapeDtypeStruct((2, NM_PAD, H), jnp.float32),
        mesh=_sc_mesh(),
        scratch_types=[
            pltpu.VMEM((4, CH), jnp.int32),
            pltpu.VMEM((4, CH, H), jnp.float32),
            pltpu.VMEM_SHARED((NM_ACC, H), jnp.float32),
            pltpu.SemaphoreType.DMA((4,)),
            pltpu.SemaphoreType.DMA((4,)),
            pltpu.SemaphoreType.DMA((4,)),
        ],
    )
    def k(enew_hbm, dst_hbm, zeros_hbm, out_hbm, iv, rv, acc, sli, slr, ss):
        c = lax.axis_index("c")
        s = lax.axis_index("s")
        pltpu.sync_copy(zeros_hbm.at[pl.ds(0, 648)], acc.at[pl.ds(s * 648, 648)])
        plsc.subcore_barrier()
        base = c * (e // 2) + s * per_t

        def loads(i, p):
            off = base + i * CH
            pltpu.async_copy(dst_hbm.at[pl.ds(off, CH)], iv.at[p], sli.at[p])
            pltpu.async_copy(enew_hbm.at[pl.ds(off, CH)], rv.at[p], slr.at[p])

        def wait_loads(p):
            _drain(dst_hbm.at[pl.ds(base, CH)], iv.at[p], sli.at[p])
            _drain(enew_hbm.at[pl.ds(base, CH)], rv.at[p], slr.at[p])

        def drain_scatter(p):
            _drain(rv.at[p], acc.at[pl.ds(0, CH)], ss.at[p])

        loads(0, 0)

        @pl.loop(0, nch // 4)
        def _(j):
            for k4 in range(4):
                i = j * 4 + k4
                nxt = (k4 + 1) % 4
                wait_loads(k4)
                pltpu.async_copy(rv.at[k4], acc.at[iv.at[k4]], ss.at[k4], add=True)

                @pl.when(i >= 3)
                def _():
                    drain_scatter(nxt)

                @pl.when(i + 1 < nch)
                def _():
                    loads(i + 1, nxt)

        drain_scatter(1)
        drain_scatter(2)
        drain_scatter(3)
        plsc.subcore_barrier()
        pltpu.sync_copy(acc.at[pl.ds(s * 640, 640)],
                        out_hbm.at[c, pl.ds(s * 640, 640)])

    return k(enew, dst, zeros_rows)


def _sc_scatter_grid(enew, dst, zeros_rows):
    e = dst.shape[0]
    per_t = e // 16
    CH = 32
    nch = per_t // CH

    @functools.partial(
        pl.kernel,
        out_type=jax.ShapeDtypeStruct((NG_PAD, H), jnp.float32),
        mesh=_sc_mesh(),
        scratch_types=[
            pltpu.VMEM((4, CH), jnp.int32),
            pltpu.VMEM((4, CH, H), jnp.float32),
            pltpu.VMEM_SHARED((ACC_G, H), jnp.float32),
            pltpu.SemaphoreType.DMA((4,)),
            pltpu.SemaphoreType.DMA((4,)),
            pltpu.SemaphoreType.DMA((4,)),
        ],
    )
    def k(enew_hbm, dst_hbm, zeros_hbm, out_hbm, iv, rv, acc, sli, slr, ss):
        c = lax.axis_index("c")
        s = lax.axis_index("s")

        def loads(i, p):
            off = s * per_t + i * CH
            pltpu.async_copy(dst_hbm.at[pl.ds(off, CH)], iv.at[p], sli.at[p])
            pltpu.async_copy(enew_hbm.at[pl.ds(off, CH)], rv.at[p], slr.at[p])

        def wait_loads(p):
            _drain(dst_hbm.at[pl.ds(0, CH)], iv.at[p], sli.at[p])
            _drain(enew_hbm.at[pl.ds(0, CH)], rv.at[p], slr.at[p])

        def drain_scatter(p):
            _drain(rv.at[p], acc.at[pl.ds(0, CH)], ss.at[p])

        for w in range(NWIN):
            base_row = w * WIN

            @pl.when(c == (w % 2))
            def _():
                pltpu.sync_copy(zeros_hbm.at[pl.ds(0, 792)],
                                acc.at[pl.ds(s * 792, 792)])
                plsc.subcore_barrier()
                loads(0, 0)

                @pl.loop(0, nch // 4)
                def _(j):
                    for k4 in range(4):
                        i = j * 4 + k4
                        nxt = (k4 + 1) % 4
                        wait_loads(k4)
                        for kk in range(2):
                            lv = iv[k4, pl.ds(kk * 16, 16)]
                            inw = (lv >= base_row) & (lv < base_row + WIN)
                            iv[k4, pl.ds(kk * 16, 16)] = jnp.where(
                                inw, lv - base_row, WIN)
                        pltpu.async_copy(rv.at[k4], acc.at[iv.at[k4]],
                                         ss.at[k4], add=True)

                        @pl.when(i >= 3)
                        def _():
                            drain_scatter(nxt)

                        @pl.when(i + 1 < nch)
                        def _():
                            loads(i + 1, nxt)

                drain_scatter(1)
                drain_scatter(2)
                drain_scatter(3)
                plsc.subcore_barrier()
                pltpu.sync_copy(acc.at[pl.ds(s * 784, 784)],
                                out_hbm.at[pl.ds(base_row + s * 784, 784)])
                plsc.subcore_barrier()

    return k(enew, dst, zeros_rows)


def _pad_rows(x, n):
    return jnp.pad(x, ((0, n - x.shape[0]), (0, 0)))


def _pad_idx(idx, n, fill):
    idx = idx.astype(jnp.int32)
    return jnp.pad(idx, (0, n - idx.shape[0]), constant_values=fill)


def kernel(input_grid_features, mesh_node_features, mesh2mesh_edge_features,
           grid2mesh_edge_features, mesh2grid_edge_features,
           m2m_src, m2m_dst, g2m_src, g2m_dst, m2g_src, m2g_dst, params):
    P = params
    grid_in = input_grid_features[0] if input_grid_features.ndim == 3 \
        else input_grid_features

    E_M2M = m2m_src.shape[0]
    E_G2M = g2m_src.shape[0]
    E_M2G = m2g_src.shape[0]
    EM = -(-E_M2M // 16384) * 16384
    EG = -(-E_G2M // 16384) * 16384
    ED = -(-E_M2G // 16384) * 16384

    gi = _pad_rows(grid_in, NG_PAD)
    mi = jnp.pad(mesh_node_features,
                 ((0, NM_PAD - NM), (0, 8 - mesh_node_features.shape[1])))
    m2m_e = jnp.pad(mesh2mesh_edge_features, ((0, EM - E_M2M), (0, 4)))
    g2m_e = jnp.pad(grid2mesh_edge_features, ((0, EG - E_G2M), (0, 4)))
    m2g_e = jnp.pad(mesh2grid_edge_features, ((0, ED - E_M2G), (0, 4)))

    m2m_src_g = _pad_idx(m2m_src, EM, 0)
    m2m_dst_g = _pad_idx(m2m_dst, EM, 0)
    m2m_dst_s = _pad_idx(m2m_dst, EM, NM_PAD)
    g2m_src_g = _pad_idx(g2m_src, EG, 0)
    g2m_dst_g = _pad_idx(g2m_dst, EG, 0)
    g2m_dst_s = _pad_idx(g2m_dst, EG, NM_PAD)
    m2g_src_g = _pad_idx(m2g_src, ED, 0)
    m2g_dst_g = _pad_idx(m2g_dst, ED, 0)
    m2g_dst_s = _pad_idx(m2g_dst, ED, GARB_FAR)

    zeros_rows = jnp.zeros((1024, H), jnp.float32)

    grid = _mlp_rows(gi, P["emb_grid"])
    mesh = _mlp_rows(mi, P["emb_mesh"],
                     w1=jnp.pad(P["emb_mesh"]["W1"], ((0, 5), (0, 0))))
    w1p4 = lambda p: jnp.pad(p["W1"], ((0, 4), (0, 0)))
    m2m = _mlp_rows(m2m_e, P["emb_m2m"], w1=w1p4(P["emb_m2m"]))
    g2m = _mlp_rows(g2m_e, P["emb_g2m"], w1=w1p4(P["emb_g2m"]))
    m2g = _mlp_rows(m2g_e, P["emb_m2g"], w1=w1p4(P["emb_m2g"]))

    w1 = P["enc_edge"]["W1"]
    ts = _mm(grid, w1[H:2 * H])
    td = _mm(mesh, w1[2 * H:])
    ga, gb = _sc_gather2(ts, td, g2m_src_g, g2m_dst_g)
    g2m = _edge_update(g2m, ga, gb, w1[:H], P["enc_edge"])
    parts = _sc_scatter_mesh(g2m, g2m_dst_s, zeros_rows)
    mesh = _node_update(mesh, parts[0], parts[1], P["enc_node"])
    grid = _mlp_rows(grid, P["enc_grid"], res=True)

    for pe, pn in zip(P["proc_edge"], P["proc_node"]):
        w1 = pe["W1"]
        ts = _mm(mesh, w1[H:2 * H])
        td = _mm(mesh, w1[2 * H:])
        ga, gb = _sc_gather2(ts, td, m2m_src_g, m2m_dst_g)
        m2m = _edge_update(m2m, ga, gb, w1[:H], pe)
        parts = _sc_scatter_mesh(m2m, m2m_dst_s, zeros_rows)
        mesh = _node_update(mesh, parts[0], parts[1], pn)

    w1 = P["dec_edge"]["W1"]
    ts = _mm(mesh, w1[H:2 * H])
    td = _mm(grid, w1[2 * H:])
    ga, gb = _sc_gather2(ts, td, m2g_src_g, m2g_dst_g)
    m2g = _edge_update(m2g, ga, gb, w1[:H], P["dec_edge"])
    agg = _sc_scatter_grid(m2g, m2g_dst_s, zeros_rows)
    grid = _node_update(grid, agg, None, P["dec_node"])

    out = _mlp_rows(grid, P["final"], norm=False)
    return out[:NG]


# speedup vs baseline: 1.0007x; 1.0007x over previous
import functools

import jax
import jax.numpy as jnp
from jax import lax
from jax.experimental import pallas as pl
from jax.experimental.pallas import tpu as pltpu
from jax.experimental.pallas import tpu_sc as plsc

H = 128
RB = 512

NG = 100000
NM = 10000
NG_PAD = 100352
NM_PAD = 10240
NM_ACC = 10368
WIN = 12544
NWIN = 8
ACC_G = 12672
GARB_FAR = 1 << 20


def _ln(y, g, beta):
    mu = jnp.mean(y, axis=-1, keepdims=True)
    var = jnp.mean((y - mu) ** 2, axis=-1, keepdims=True)
    return (y - mu) * lax.rsqrt(var + 1e-5) * g + beta


def _silu(x):
    return x * jax.nn.sigmoid(x)


def _dot(a, b):
    return jnp.dot(a, b, preferred_element_type=jnp.float32)


def _mlp_body(x_ref, w1_ref, b1_ref, w2_ref, b2_ref, g_ref, beta_ref, o_ref,
              *, norm, res):
    x = x_ref[...]
    h = _silu(_dot(x, w1_ref[...]) + b1_ref[...])
    y = _dot(h, w2_ref[...]) + b2_ref[...]
    if norm:
        y = _ln(y, g_ref[...], beta_ref[...])
    if res:
        y = y + x
    o_ref[...] = y


def _mlp_rows(x, p, *, norm=True, res=False, w1=None):
    n, din = x.shape
    w1 = p["W1"] if w1 is None else w1
    dh = w1.shape[1]
    w2 = p["W2"]
    dout = w2.shape[1]
    if norm:
        g, beta = p["g"].reshape(1, -1), p["beta"].reshape(1, -1)
    else:
        g = jnp.ones((1, dout), jnp.float32)
        beta = jnp.zeros((1, dout), jnp.float32)
    return pl.pallas_call(
        functools.partial(_mlp_body, norm=norm, res=res),
        grid=(n // RB,),
        in_specs=[
            pl.BlockSpec((RB, din), lambda i: (i, 0)),
            pl.BlockSpec((din, dh), lambda i: (0, 0)),
            pl.BlockSpec((1, dh), lambda i: (0, 0)),
            pl.BlockSpec((dh, dout), lambda i: (0, 0)),
            pl.BlockSpec((1, dout), lambda i: (0, 0)),
            pl.BlockSpec((1, dout), lambda i: (0, 0)),
            pl.BlockSpec((1, dout), lambda i: (0, 0)),
        ],
        out_specs=pl.BlockSpec((RB, dout), lambda i: (i, 0)),
        out_shape=jax.ShapeDtypeStruct((n, dout), jnp.float32),
    )(x, w1, p["b1"].reshape(1, -1), w2, p["b2"].reshape(1, -1), g, beta)


def _edge_body(e_ref, ga_ref, gb_ref, w1_ref, b1_ref, w2_ref, b2_ref,
               g_ref, beta_ref, o_ref):
    e = e_ref[...]
    h = _silu(_dot(e, w1_ref[...]) + ga_ref[...] + gb_ref[...] + b1_ref[...])
    y = _dot(h, w2_ref[...]) + b2_ref[...]
    o_ref[...] = e + _ln(y, g_ref[...], beta_ref[...])


def _edge_update(e, ga, gb, w1e, p):
    n = e.shape[0]
    return pl.pallas_call(
        _edge_body,
        grid=(n // RB,),
        in_specs=[
            pl.BlockSpec((RB, H), lambda i: (i, 0)),
            pl.BlockSpec((RB, H), lambda i: (i, 0)),
            pl.BlockSpec((RB, H), lambda i: (i, 0)),
            pl.BlockSpec((H, H), lambda i: (0, 0)),
            pl.BlockSpec((1, H), lambda i: (0, 0)),
            pl.BlockSpec((H, H), lambda i: (0, 0)),
            pl.BlockSpec((1, H), lambda i: (0, 0)),
            pl.BlockSpec((1, H), lambda i: (0, 0)),
            pl.BlockSpec((1, H), lambda i: (0, 0)),
        ],
        out_specs=pl.BlockSpec((RB, H), lambda i: (i, 0)),
        out_shape=jax.ShapeDtypeStruct((n, H), jnp.float32),
    )(e, ga, gb, w1e, p["b1"].reshape(1, -1), p["W2"],
      p["b2"].reshape(1, -1), p["g"].reshape(1, -1), p["beta"].reshape(1, -1))


def _node_body(x_ref, a1_ref, a2_ref, w1x_ref, w1a_ref, b1_ref, w2_ref,
               b2_ref, g_ref, beta_ref, o_ref, *, two_agg):
    x = x_ref[...]
    agg = a1_ref[...]
    if two_agg:
        agg = agg + a2_ref[...]
    h = _silu(_dot(x, w1x_ref[...]) + _dot(agg, w1a_ref[...]) + b1_ref[...])
    y = _dot(h, w2_ref[...]) + b2_ref[...]
    o_ref[...] = x + _ln(y, g_ref[...], beta_ref[...])


def _node_update(x, a1, a2, p):
    n = x.shape[0]
    two_agg = a2 is not None
    if not two_agg:
        a2 = a1
    w1 = p["W1"]
    return pl.pallas_call(
        functools.partial(_node_body, two_agg=two_agg),
        grid=(n // RB,),
        in_specs=[pl.BlockSpec((RB, H), lambda i: (i, 0))] * 3
        + [
            pl.BlockSpec((H, H), lambda i: (0, 0)),
            pl.BlockSpec((H, H), lambda i: (0, 0)),
            pl.BlockSpec((1, H), lambda i: (0, 0)),
            pl.BlockSpec((H, H), lambda i: (0, 0)),
            pl.BlockSpec((1, H), lambda i: (0, 0)),
            pl.BlockSpec((1, H), lambda i: (0, 0)),
            pl.BlockSpec((1, H), lambda i: (0, 0)),
        ],
        out_specs=pl.BlockSpec((RB, H), lambda i: (i, 0)),
        out_shape=jax.ShapeDtypeStruct((n, H), jnp.float32),
    )(x, a1, a2, w1[:H], w1[H:], p["b1"].reshape(1, -1), p["W2"],
      p["b2"].reshape(1, -1), p["g"].reshape(1, -1), p["beta"].reshape(1, -1))


def _mm_body(x_ref, w_ref, o_ref):
    o_ref[...] = _dot(x_ref[...], w_ref[...])


def _mm(x, w):
    n = x.shape[0]
    dout = w.shape[1]
    return pl.pallas_call(
        _mm_body,
        grid=(n // RB,),
        in_specs=[
            pl.BlockSpec((RB, H), lambda i: (i, 0)),
            pl.BlockSpec((H, dout), lambda i: (0, 0)),
        ],
        out_specs=pl.BlockSpec((RB, dout), lambda i: (i, 0)),
        out_shape=jax.ShapeDtypeStruct((n, dout), jnp.float32),
    )(x, w)


def _sc_mesh():
    return plsc.VectorSubcoreMesh(core_axis_name="c", subcore_axis_name="s")


def _drain(src, dst, sem):
    pltpu.make_async_copy(src, dst, sem).wait()


def _sc_gather2(table_a, table_b, idx_a, idx_b):
    (e,) = idx_a.shape
    per_w = e // 32
    CH = 128
    nch = per_w // CH

    @functools.partial(
        pl.kernel,
        out_type=(jax.ShapeDtypeStruct((e, H), jnp.float32),
                  jax.ShapeDtypeStruct((e, H), jnp.float32)),
        mesh=_sc_mesh(),
        scratch_types=[
            pltpu.VMEM((2, 4, CH), jnp.int32),
            pltpu.VMEM((2, 2, CH, H), jnp.float32),
            pltpu.SemaphoreType.DMA((2, 4)),
            pltpu.SemaphoreType.DMA((2, 2)),
            pltpu.SemaphoreType.DMA((2, 2)),
        ],
    )
    def k(ta, tb, ia_h, ib_h, oa, ob, iv, rv, si, sg, sw):
        c = lax.axis_index("c")
        s = lax.axis_index("s")
        base = (s * 2 + c) * per_w

        def idx_load(i, isl):
            off = base + i * CH
            pltpu.async_copy(ia_h.at[pl.ds(off, CH)], iv.at[0, isl], si.at[0, isl])
            pltpu.async_copy(ib_h.at[pl.ds(off, CH)], iv.at[1, isl], si.at[1, isl])

        def wait_idx(isl):
            _drain(ia_h.at[pl.ds(base, CH)], iv.at[0, isl], si.at[0, isl])
            _drain(ib_h.at[pl.ds(base, CH)], iv.at[1, isl], si.at[1, isl])

        def wait_gather(rs):
            _drain(ta.at[iv.at[0, 0]], rv.at[0, rs], sg.at[0, rs])
            _drain(tb.at[iv.at[1, 0]], rv.at[1, rs], sg.at[1, rs])

        def start_wb(i, rs):
            off = base + i * CH
            pltpu.async_copy(rv.at[0, rs], oa.at[pl.ds(off, CH)], sw.at[0, rs])
            pltpu.async_copy(rv.at[1, rs], ob.at[pl.ds(off, CH)], sw.at[1, rs])

        def wait_wb(rs):
            _drain(rv.at[0, rs], oa.at[pl.ds(base, CH)], sw.at[0, rs])
            _drain(rv.at[1, rs], ob.at[pl.ds(base, CH)], sw.at[1, rs])

        idx_load(0, 0)
        idx_load(1, 1)

        @pl.loop(0, nch // 4)
        def _(j):
            for k4 in range(4):
                rs = k4 % 2
                i = j * 4 + k4

                @pl.when(i >= 2)
                def _():
                    wait_wb(rs)

                wait_idx(k4)
                pltpu.async_copy(ta.at[iv.at[0, k4]], rv.at[0, rs], sg.at[0, rs])
                pltpu.async_copy(tb.at[iv.at[1, k4]], rv.at[1, rs], sg.at[1, rs])

                @pl.when(i + 2 < nch)
                def _():
                    idx_load(i + 2, (k4 + 2) % 4)

                @pl.when(i >= 1)
                def _():
                    wait_gather(1 - rs)
                    start_wb(i - 1, 1 - rs)

        wait_gather(1)
        start_wb(nch - 1, 1)
        wait_wb(0)
        wait_wb(1)

    return k(table_a, table_b, idx_a, idx_b)


def _sc_scatter_mesh(enew, dst, zeros_rows):
    e = dst.shape[0]
    per_t = e // 32
    CH = 64
    nch = per_t // CH

    @functools.partial(
        pl.kernel,
        out_type=jax.ShapeDtypeStruct((2, NM_PAD, H), jnp.float32),
        mesh=_sc_mesh(),
        scratch_types=[
            pltpu.VMEM((4, CH), jnp.int32),
            pltpu.VMEM((4, CH, H), jnp.float32),
            pltpu.VMEM_SHARED((NM_ACC, H), jnp.float32),
            pltpu.SemaphoreType.DMA((4,)),
            pltpu.SemaphoreType.DMA((4,)),
            pltpu.SemaphoreType.DMA((4,)),
        ],
    )
    def k(enew_hbm, dst_hbm, zeros_hbm, out_hbm, iv, rv, acc, sli, slr, ss):
        c = lax.axis_index("c")
        s = lax.axis_index("s")
        pltpu.sync_copy(zeros_hbm.at[pl.ds(0, 648)], acc.at[pl.ds(s * 648, 648)])
        plsc.subcore_barrier()
        base = c * (e // 2) + s * per_t

        def loads(i, p):
            off = base + i * CH
            pltpu.async_copy(dst_hbm.at[pl.ds(off, CH)], iv.at[p], sli.at[p])
            pltpu.async_copy(enew_hbm.at[pl.ds(off, CH)], rv.at[p], slr.at[p])

        def wait_loads(p):
            _drain(dst_hbm.at[pl.ds(base, CH)], iv.at[p], sli.at[p])
            _drain(enew_hbm.at[pl.ds(base, CH)], rv.at[p], slr.at[p])

        def drain_scatter(p):
            _drain(rv.at[p], acc.at[pl.ds(0, CH)], ss.at[p])

        loads(0, 0)

        @pl.loop(0, nch // 4)
        def _(j):
            for k4 in range(4):
                i = j * 4 + k4
                nxt = (k4 + 1) % 4
                wait_loads(k4)
                pltpu.async_copy(rv.at[k4], acc.at[iv.at[k4]], ss.at[k4], add=True)

                @pl.when(i >= 3)
                def _():
                    drain_scatter(nxt)

                @pl.when(i + 1 < nch)
                def _():
                    loads(i + 1, nxt)

        drain_scatter(1)
        drain_scatter(2)
        drain_scatter(3)
        plsc.subcore_barrier()
        pltpu.sync_copy(acc.at[pl.ds(s * 640, 640)],
                        out_hbm.at[c, pl.ds(s * 640, 640)])

    return k(enew, dst, zeros_rows)


def _sc_scatter_grid(enew, dst, zeros_rows):
    e = dst.shape[0]
    per_t = e // 16
    CH = 32
    nch = per_t // CH

    @functools.partial(
        pl.kernel,
        out_type=jax.ShapeDtypeStruct((NG_PAD, H), jnp.float32),
        mesh=_sc_mesh(),
        scratch_types=[
            pltpu.VMEM((4, CH), jnp.int32),
            pltpu.VMEM((4, CH, H), jnp.float32),
            pltpu.VMEM_SHARED((ACC_G, H), jnp.float32),
            pltpu.SemaphoreType.DMA((4,)),
            pltpu.SemaphoreType.DMA((4,)),
            pltpu.SemaphoreType.DMA((4,)),
        ],
    )
    def k(enew_hbm, dst_hbm, zeros_hbm, out_hbm, iv, rv, acc, sli, slr, ss):
        c = lax.axis_index("c")
        s = lax.axis_index("s")

        def loads(i, p):
            off = s * per_t + i * CH
            pltpu.async_copy(dst_hbm.at[pl.ds(off, CH)], iv.at[p], sli.at[p])
            pltpu.async_copy(enew_hbm.at[pl.ds(off, CH)], rv.at[p], slr.at[p])

        def wait_loads(p):
            _drain(dst_hbm.at[pl.ds(0, CH)], iv.at[p], sli.at[p])
            _drain(enew_hbm.at[pl.ds(0, CH)], rv.at[p], slr.at[p])

        def drain_scatter(p):
            _drain(rv.at[p], acc.at[pl.ds(0, CH)], ss.at[p])

        for w in range(NWIN):
            base_row = w * WIN

            @pl.when(c == (w % 2))
            def _():
                pltpu.sync_copy(zeros_hbm.at[pl.ds(0, 792)],
                                acc.at[pl.ds(s * 792, 792)])
                plsc.subcore_barrier()
                loads(0, 0)

                @pl.loop(0, nch // 4)
                def _(j):
                    for k4 in range(4):
                        i = j * 4 + k4
                        nxt = (k4 + 1) % 4
                        wait_loads(k4)
                        for kk in range(2):
                            lv = iv[k4, pl.ds(kk * 16, 16)]
                            inw = (lv >= base_row) & (lv < base_row + WIN)
                            iv[k4, pl.ds(kk * 16, 16)] = jnp.where(
                                inw, lv - base_row, WIN)
                        pltpu.async_copy(rv.at[k4], acc.at[iv.at[k4]],
                                         ss.at[k4], add=True)

                        @pl.when(i >= 3)
                        def _():
                            drain_scatter(nxt)

                        @pl.when(i + 1 < nch)
                        def _():
                            loads(i + 1, nxt)

                drain_scatter(1)
                drain_scatter(2)
                drain_scatter(3)
                plsc.subcore_barrier()
                pltpu.sync_copy(acc.at[pl.ds(s * 784, 784)],
                                out_hbm.at[pl.ds(base_row + s * 784, 784)])
                plsc.subcore_barrier()

    return k(enew, dst, zeros_rows)


def _pad_rows(x, n):
    return jnp.pad(x, ((0, n - x.shape[0]), (0, 0)))


def _pad_idx(idx, n, fill):
    idx = idx.astype(jnp.int32)
    return jnp.pad(idx, (0, n - idx.shape[0]), constant_values=fill)


def kernel(input_grid_features, mesh_node_features, mesh2mesh_edge_features,
           grid2mesh_edge_features, mesh2grid_edge_features,
           m2m_src, m2m_dst, g2m_src, g2m_dst, m2g_src, m2g_dst, params):
    P = params
    grid_in = input_grid_features[0] if input_grid_features.ndim == 3 \
        else input_grid_features

    E_M2M = m2m_src.shape[0]
    E_G2M = g2m_src.shape[0]
    E_M2G = m2g_src.shape[0]
    EM = -(-E_M2M // 16384) * 16384
    EG = -(-E_G2M // 16384) * 16384
    ED = -(-E_M2G // 16384) * 16384

    gi = _pad_rows(grid_in, NG_PAD)
    mi = jnp.pad(mesh_node_features,
                 ((0, NM_PAD - NM), (0, 8 - mesh_node_features.shape[1])))
    m2m_e = jnp.pad(mesh2mesh_edge_features, ((0, EM - E_M2M), (0, 4)))
    g2m_e = jnp.pad(grid2mesh_edge_features, ((0, EG - E_G2M), (0, 4)))
    m2g_e = jnp.pad(mesh2grid_edge_features, ((0, ED - E_M2G), (0, 4)))

    m2m_src_g = _pad_idx(m2m_src, EM, 0)
    m2m_dst_g = _pad_idx(m2m_dst, EM, 0)
    m2m_dst_s = _pad_idx(m2m_dst, EM, NM_PAD)
    g2m_src_g = _pad_idx(g2m_src, EG, 0)
    g2m_dst_g = _pad_idx(g2m_dst, EG, 0)
    g2m_dst_s = _pad_idx(g2m_dst, EG, NM_PAD)
    m2g_src_g = _pad_idx(m2g_src, ED, 0)
    m2g_dst_g = _pad_idx(m2g_dst, ED, 0)
    m2g_dst_s = _pad_idx(m2g_dst, ED, GARB_FAR)

    zeros_rows = jnp.zeros((1024, H), jnp.float32)

    grid = _mlp_rows(gi, P["emb_grid"])
    mesh = _mlp_rows(mi, P["emb_mesh"],
                     w1=jnp.pad(P["emb_mesh"]["W1"], ((0, 5), (0, 0))))
    w1p4 = lambda p: jnp.pad(p["W1"], ((0, 4), (0, 0)))
    m2m = _mlp_rows(m2m_e, P["emb_m2m"], w1=w1p4(P["emb_m2m"]))
    g2m = _mlp_rows(g2m_e, P["emb_g2m"], w1=w1p4(P["emb_g2m"]))
    m2g = _mlp_rows(m2g_e, P["emb_m2g"], w1=w1p4(P["emb_m2g"]))

    w1 = P["enc_edge"]["W1"]
    ts = _mm(grid, w1[H:2 * H])
    td = _mm(mesh, w1[2 * H:])
    ga, gb = _sc_gather2(ts, td, g2m_src_g, g2m_dst_g)
    g2m = _edge_update(g2m, ga, gb, w1[:H], P["enc_edge"])
    parts = _sc_scatter_mesh(g2m, g2m_dst_s, zeros_rows)
    mesh = _node_update(mesh, parts[0], parts[1], P["enc_node"])
    grid = _mlp_rows(grid, P["enc_grid"], res=True)

    for pe, pn in zip(P["proc_edge"], P["proc_node"]):
        w1 = pe["W1"]
        ts = _mm(mesh, w1[H:2 * H])
        td = _mm(mesh, w1[2 * H:])
        ga, gb = _sc_gather2(ts, td, m2m_src_g, m2m_dst_g)
        m2m = _edge_update(m2m, ga, gb, w1[:H], pe)
        parts = _sc_scatter_mesh(m2m, m2m_dst_s, zeros_rows)
        mesh = _node_update(mesh, parts[0], parts[1], pn)

    w1 = P["dec_edge"]["W1"]
    ts = _mm(mesh, w1[H:2 * H])
    td = _mm(grid, w1[2 * H:])
    ga, gb = _sc_gather2(ts, td, m2g_src_g, m2g_dst_g)
    m2g = _edge_update(m2g, ga, gb, w1[:H], P["dec_edge"])
    agg = _sc_scatter_grid(m2g, m2g_dst_s, zeros_rows)
    grid = _node_update(grid, agg, None, P["dec_node"])

    out = _mlp_rows(grid, P["final"], norm=False)
    return out[:NG]


# speedup vs baseline: 1.0655x; 1.0648x over previous
import functools

import jax
import jax.numpy as jnp
from jax import lax
from jax.experimental import pallas as pl
from jax.experimental.pallas import tpu as pltpu
from jax.experimental.pallas import tpu_sc as plsc

H = 128
RB = 512

NG = 100000
NM = 10000
NG_PAD = 100352
NM_PAD = 10240
NM_ACC = 10368
WIN = 12544
NWIN = 8
ACC_G = 12672
GARB_FAR = 1 << 20


def _ln(y, g, beta):
    mu = jnp.mean(y, axis=-1, keepdims=True)
    var = jnp.mean((y - mu) ** 2, axis=-1, keepdims=True)
    return (y - mu) * lax.rsqrt(var + 1e-5) * g + beta


def _silu(x):
    return x * jax.nn.sigmoid(x)


def _dot(a, b):
    return jnp.dot(a, b, preferred_element_type=jnp.float32)


def _mlp_body(x_ref, w1_ref, b1_ref, w2_ref, b2_ref, g_ref, beta_ref, o_ref,
              *, norm, res):
    x = x_ref[...]
    h = _silu(_dot(x, w1_ref[...]) + b1_ref[...])
    y = _dot(h, w2_ref[...]) + b2_ref[...]
    if norm:
        y = _ln(y, g_ref[...], beta_ref[...])
    if res:
        y = y + x
    o_ref[...] = y


def _mlp_rows(x, p, *, norm=True, res=False, w1=None):
    n, din = x.shape
    w1 = p["W1"] if w1 is None else w1
    dh = w1.shape[1]
    w2 = p["W2"]
    dout = w2.shape[1]
    if norm:
        g, beta = p["g"].reshape(1, -1), p["beta"].reshape(1, -1)
    else:
        g = jnp.ones((1, dout), jnp.float32)
        beta = jnp.zeros((1, dout), jnp.float32)
    return pl.pallas_call(
        functools.partial(_mlp_body, norm=norm, res=res),
        grid=(n // RB,),
        in_specs=[
            pl.BlockSpec((RB, din), lambda i: (i, 0)),
            pl.BlockSpec((din, dh), lambda i: (0, 0)),
            pl.BlockSpec((1, dh), lambda i: (0, 0)),
            pl.BlockSpec((dh, dout), lambda i: (0, 0)),
            pl.BlockSpec((1, dout), lambda i: (0, 0)),
            pl.BlockSpec((1, dout), lambda i: (0, 0)),
            pl.BlockSpec((1, dout), lambda i: (0, 0)),
        ],
        out_specs=pl.BlockSpec((RB, dout), lambda i: (i, 0)),
        out_shape=jax.ShapeDtypeStruct((n, dout), jnp.float32),
    )(x, w1, p["b1"].reshape(1, -1), w2, p["b2"].reshape(1, -1), g, beta)


def _edge_body(e_ref, ga_ref, gb_ref, w1_ref, b1_ref, w2_ref, b2_ref,
               g_ref, beta_ref, o_ref):
    e = e_ref[...]
    h = _silu(_dot(e, w1_ref[...]) + ga_ref[...] + gb_ref[...] + b1_ref[...])
    y = _dot(h, w2_ref[...]) + b2_ref[...]
    o_ref[...] = e + _ln(y, g_ref[...], beta_ref[...])


def _edge_update(e, ga, gb, w1e, p):
    n = e.shape[0]
    return pl.pallas_call(
        _edge_body,
        grid=(n // RB,),
        in_specs=[
            pl.BlockSpec((RB, H), lambda i: (i, 0)),
            pl.BlockSpec((RB, H), lambda i: (i, 0)),
            pl.BlockSpec((RB, H), lambda i: (i, 0)),
            pl.BlockSpec((H, H), lambda i: (0, 0)),
            pl.BlockSpec((1, H), lambda i: (0, 0)),
            pl.BlockSpec((H, H), lambda i: (0, 0)),
            pl.BlockSpec((1, H), lambda i: (0, 0)),
            pl.BlockSpec((1, H), lambda i: (0, 0)),
            pl.BlockSpec((1, H), lambda i: (0, 0)),
        ],
        out_specs=pl.BlockSpec((RB, H), lambda i: (i, 0)),
        out_shape=jax.ShapeDtypeStruct((n, H), jnp.float32),
    )(e, ga, gb, w1e, p["b1"].reshape(1, -1), p["W2"],
      p["b2"].reshape(1, -1), p["g"].reshape(1, -1), p["beta"].reshape(1, -1))


def _node_body(x_ref, a1_ref, a2_ref, w1x_ref, w1a_ref, b1_ref, w2_ref,
               b2_ref, g_ref, beta_ref, o_ref, *, two_agg):
    x = x_ref[...]
    agg = a1_ref[...]
    if two_agg:
        agg = agg + a2_ref[...]
    h = _silu(_dot(x, w1x_ref[...]) + _dot(agg, w1a_ref[...]) + b1_ref[...])
    y = _dot(h, w2_ref[...]) + b2_ref[...]
    o_ref[...] = x + _ln(y, g_ref[...], beta_ref[...])


def _node_update(x, a1, a2, p):
    n = x.shape[0]
    two_agg = a2 is not None
    if not two_agg:
        a2 = a1
    w1 = p["W1"]
    return pl.pallas_call(
        functools.partial(_node_body, two_agg=two_agg),
        grid=(n // RB,),
        in_specs=[pl.BlockSpec((RB, H), lambda i: (i, 0))] * 3
        + [
            pl.BlockSpec((H, H), lambda i: (0, 0)),
            pl.BlockSpec((H, H), lambda i: (0, 0)),
            pl.BlockSpec((1, H), lambda i: (0, 0)),
            pl.BlockSpec((H, H), lambda i: (0, 0)),
            pl.BlockSpec((1, H), lambda i: (0, 0)),
            pl.BlockSpec((1, H), lambda i: (0, 0)),
            pl.BlockSpec((1, H), lambda i: (0, 0)),
        ],
        out_specs=pl.BlockSpec((RB, H), lambda i: (i, 0)),
        out_shape=jax.ShapeDtypeStruct((n, H), jnp.float32),
    )(x, a1, a2, w1[:H], w1[H:], p["b1"].reshape(1, -1), p["W2"],
      p["b2"].reshape(1, -1), p["g"].reshape(1, -1), p["beta"].reshape(1, -1))


def _mm_body(x_ref, w_ref, o_ref):
    o_ref[...] = _dot(x_ref[...], w_ref[...])


def _mm(x, w):
    n = x.shape[0]
    dout = w.shape[1]
    return pl.pallas_call(
        _mm_body,
        grid=(n // RB,),
        in_specs=[
            pl.BlockSpec((RB, H), lambda i: (i, 0)),
            pl.BlockSpec((H, dout), lambda i: (0, 0)),
        ],
        out_specs=pl.BlockSpec((RB, dout), lambda i: (i, 0)),
        out_shape=jax.ShapeDtypeStruct((n, dout), jnp.float32),
    )(x, w)


def _sc_mesh():
    return plsc.VectorSubcoreMesh(core_axis_name="c", subcore_axis_name="s")


def _drain(src, dst, sem):
    pltpu.make_async_copy(src, dst, sem).wait()


def _sc_gather2(table_a, table_b, idx_a, idx_b):
    (e,) = idx_a.shape
    per_w = e // 32
    CH = 128
    nch = per_w // CH

    @functools.partial(
        pl.kernel,
        out_type=(jax.ShapeDtypeStruct((e, H), jnp.float32),
                  jax.ShapeDtypeStruct((e, H), jnp.float32)),
        mesh=_sc_mesh(),
        scratch_types=[
            pltpu.VMEM((2, 4, CH), jnp.int32),
            pltpu.VMEM((2, 2, CH, H), jnp.float32),
            pltpu.SemaphoreType.DMA((2, 4)),
            pltpu.SemaphoreType.DMA((2, 2)),
            pltpu.SemaphoreType.DMA((2, 2)),
        ],
    )
    def k(ta, tb, ia_h, ib_h, oa, ob, iv, rv, si, sg, sw):
        c = lax.axis_index("c")
        s = lax.axis_index("s")
        base = (s * 2 + c) * per_w

        def idx_load(i, isl):
            off = base + i * CH
            pltpu.async_copy(ia_h.at[pl.ds(off, CH)], iv.at[0, isl], si.at[0, isl])
            pltpu.async_copy(ib_h.at[pl.ds(off, CH)], iv.at[1, isl], si.at[1, isl])

        def wait_idx(isl):
            _drain(ia_h.at[pl.ds(base, CH)], iv.at[0, isl], si.at[0, isl])
            _drain(ib_h.at[pl.ds(base, CH)], iv.at[1, isl], si.at[1, isl])

        def wait_gather(rs):
            _drain(ta.at[iv.at[0, 0]], rv.at[0, rs], sg.at[0, rs])
            _drain(tb.at[iv.at[1, 0]], rv.at[1, rs], sg.at[1, rs])

        def start_wb(i, rs):
            off = base + i * CH
            pltpu.async_copy(rv.at[0, rs], oa.at[pl.ds(off, CH)], sw.at[0, rs])
            pltpu.async_copy(rv.at[1, rs], ob.at[pl.ds(off, CH)], sw.at[1, rs])

        def wait_wb(rs):
            _drain(rv.at[0, rs], oa.at[pl.ds(base, CH)], sw.at[0, rs])
            _drain(rv.at[1, rs], ob.at[pl.ds(base, CH)], sw.at[1, rs])

        idx_load(0, 0)
        idx_load(1, 1)

        @pl.loop(0, nch // 4)
        def _(j):
            for k4 in range(4):
                rs = k4 % 2
                i = j * 4 + k4

                @pl.when(i >= 2)
                def _():
                    wait_wb(rs)

                wait_idx(k4)
                pltpu.async_copy(ta.at[iv.at[0, k4]], rv.at[0, rs], sg.at[0, rs])
                pltpu.async_copy(tb.at[iv.at[1, k4]], rv.at[1, rs], sg.at[1, rs])

                @pl.when(i + 2 < nch)
                def _():
                    idx_load(i + 2, (k4 + 2) % 4)

                @pl.when(i >= 1)
                def _():
                    wait_gather(1 - rs)
                    start_wb(i - 1, 1 - rs)

        wait_gather(1)
        start_wb(nch - 1, 1)
        wait_wb(0)
        wait_wb(1)

    return k(table_a, table_b, idx_a, idx_b)


def _sc_scatter_mesh(enew, dst, zeros_rows):
    e = dst.shape[0]
    per_t = e // 32
    CH = 64
    nch = per_t // CH

    @functools.partial(
        pl.kernel,
        out_type=jax.ShapeDtypeStruct((2, NM_PAD, H), jnp.float32),
        mesh=_sc_mesh(),
        scratch_types=[
            pltpu.VMEM((4, CH), jnp.int32),
            pltpu.VMEM((4, CH, H), jnp.float32),
            pltpu.VMEM_SHARED((NM_ACC, H), jnp.float32),
            pltpu.SemaphoreType.DMA((4,)),
            pltpu.SemaphoreType.DMA((4,)),
            pltpu.SemaphoreType.DMA((4,)),
        ],
    )
    def k(enew_hbm, dst_hbm, zeros_hbm, out_hbm, iv, rv, acc, sli, slr, ss):
        c = lax.axis_index("c")
        s = lax.axis_index("s")
        pltpu.sync_copy(zeros_hbm.at[pl.ds(0, 648)], acc.at[pl.ds(s * 648, 648)])
        plsc.subcore_barrier()
        base = c * (e // 2) + s * per_t

        def loads(i, p):
            off = base + i * CH
            pltpu.async_copy(dst_hbm.at[pl.ds(off, CH)], iv.at[p], sli.at[p])
            pltpu.async_copy(enew_hbm.at[pl.ds(off, CH)], rv.at[p], slr.at[p])

        def wait_loads(p):
            _drain(dst_hbm.at[pl.ds(base, CH)], iv.at[p], sli.at[p])
            _drain(enew_hbm.at[pl.ds(base, CH)], rv.at[p], slr.at[p])

        def drain_scatter(p):
            _drain(rv.at[p], acc.at[pl.ds(0, CH)], ss.at[p])

        loads(0, 0)
        loads(1, 1)

        @pl.loop(0, nch // 4)
        def _(j):
            for k4 in range(4):
                i = j * 4 + k4
                nxt = (k4 + 2) % 4
                wait_loads(k4)
                pltpu.async_copy(rv.at[k4], acc.at[iv.at[k4]], ss.at[k4], add=True)

                @pl.when(i >= 2)
                def _():
                    drain_scatter(nxt)

                @pl.when(i + 2 < nch)
                def _():
                    loads(i + 2, nxt)

        drain_scatter(2)
        drain_scatter(3)
        plsc.subcore_barrier()
        pltpu.sync_copy(acc.at[pl.ds(s * 640, 640)],
                        out_hbm.at[c, pl.ds(s * 640, 640)])

    return k(enew, dst, zeros_rows)


def _sc_scatter_grid(enew, dst, zeros_rows):
    e = dst.shape[0]
    per_t = e // 16
    CH = 32
    nch = per_t // CH

    @functools.partial(
        pl.kernel,
        out_type=jax.ShapeDtypeStruct((NG_PAD, H), jnp.float32),
        mesh=_sc_mesh(),
        scratch_types=[
            pltpu.VMEM((4, CH), jnp.int32),
            pltpu.VMEM((4, CH, H), jnp.float32),
            pltpu.VMEM_SHARED((ACC_G, H), jnp.float32),
            pltpu.SemaphoreType.DMA((4,)),
            pltpu.SemaphoreType.DMA((4,)),
            pltpu.SemaphoreType.DMA((4,)),
        ],
    )
    def k(enew_hbm, dst_hbm, zeros_hbm, out_hbm, iv, rv, acc, sli, slr, ss):
        c = lax.axis_index("c")
        s = lax.axis_index("s")

        def loads(i, p):
            off = s * per_t + i * CH
            pltpu.async_copy(dst_hbm.at[pl.ds(off, CH)], iv.at[p], sli.at[p])
            pltpu.async_copy(enew_hbm.at[pl.ds(off, CH)], rv.at[p], slr.at[p])

        def wait_loads(p):
            _drain(dst_hbm.at[pl.ds(0, CH)], iv.at[p], sli.at[p])
            _drain(enew_hbm.at[pl.ds(0, CH)], rv.at[p], slr.at[p])

        def drain_scatter(p):
            _drain(rv.at[p], acc.at[pl.ds(0, CH)], ss.at[p])

        for w in range(NWIN):
            base_row = w * WIN

            @pl.when(c == (w % 2))
            def _():
                pltpu.sync_copy(zeros_hbm.at[pl.ds(0, 792)],
                                acc.at[pl.ds(s * 792, 792)])
                plsc.subcore_barrier()
                loads(0, 0)
                loads(1, 1)

                @pl.loop(0, nch // 4)
                def _(j):
                    for k4 in range(4):
                        i = j * 4 + k4
                        nxt = (k4 + 2) % 4
                        wait_loads(k4)
                        for kk in range(2):
                            lv = iv[k4, pl.ds(kk * 16, 16)]
                            inw = (lv >= base_row) & (lv < base_row + WIN)
                            iv[k4, pl.ds(kk * 16, 16)] = jnp.where(
                                inw, lv - base_row, WIN)
                        pltpu.async_copy(rv.at[k4], acc.at[iv.at[k4]],
                                         ss.at[k4], add=True)

                        @pl.when(i >= 2)
                        def _():
                            drain_scatter(nxt)

                        @pl.when(i + 2 < nch)
                        def _():
                            loads(i + 2, nxt)

                drain_scatter(2)
                drain_scatter(3)
                plsc.subcore_barrier()
                pltpu.sync_copy(acc.at[pl.ds(s * 784, 784)],
                                out_hbm.at[pl.ds(base_row + s * 784, 784)])
                plsc.subcore_barrier()

    return k(enew, dst, zeros_rows)


def _pad_rows(x, n):
    return jnp.pad(x, ((0, n - x.shape[0]), (0, 0)))


def _pad_idx(idx, n, fill):
    idx = idx.astype(jnp.int32)
    return jnp.pad(idx, (0, n - idx.shape[0]), constant_values=fill)


def kernel(input_grid_features, mesh_node_features, mesh2mesh_edge_features,
           grid2mesh_edge_features, mesh2grid_edge_features,
           m2m_src, m2m_dst, g2m_src, g2m_dst, m2g_src, m2g_dst, params):
    P = params
    grid_in = input_grid_features[0] if input_grid_features.ndim == 3 \
        else input_grid_features

    E_M2M = m2m_src.shape[0]
    E_G2M = g2m_src.shape[0]
    E_M2G = m2g_src.shape[0]
    EM = -(-E_M2M // 16384) * 16384
    EG = -(-E_G2M // 16384) * 16384
    ED = -(-E_M2G // 16384) * 16384

    gi = _pad_rows(grid_in, NG_PAD)
    mi = jnp.pad(mesh_node_features,
                 ((0, NM_PAD - NM), (0, 8 - mesh_node_features.shape[1])))
    m2m_e = jnp.pad(mesh2mesh_edge_features, ((0, EM - E_M2M), (0, 4)))
    g2m_e = jnp.pad(grid2mesh_edge_features, ((0, EG - E_G2M), (0, 4)))
    m2g_e = jnp.pad(mesh2grid_edge_features, ((0, ED - E_M2G), (0, 4)))

    m2m_src_g = _pad_idx(m2m_src, EM, 0)
    m2m_dst_g = _pad_idx(m2m_dst, EM, 0)
    m2m_dst_s = _pad_idx(m2m_dst, EM, NM_PAD)
    g2m_src_g = _pad_idx(g2m_src, EG, 0)
    g2m_dst_g = _pad_idx(g2m_dst, EG, 0)
    g2m_dst_s = _pad_idx(g2m_dst, EG, NM_PAD)
    m2g_src_g = _pad_idx(m2g_src, ED, 0)
    m2g_dst_g = _pad_idx(m2g_dst, ED, 0)
    m2g_dst_s = _pad_idx(m2g_dst, ED, GARB_FAR)

    zeros_rows = jnp.zeros((1024, H), jnp.float32)

    grid = _mlp_rows(gi, P["emb_grid"])
    mesh = _mlp_rows(mi, P["emb_mesh"],
                     w1=jnp.pad(P["emb_mesh"]["W1"], ((0, 5), (0, 0))))
    w1p4 = lambda p: jnp.pad(p["W1"], ((0, 4), (0, 0)))
    m2m = _mlp_rows(m2m_e, P["emb_m2m"], w1=w1p4(P["emb_m2m"]))
    g2m = _mlp_rows(g2m_e, P["emb_g2m"], w1=w1p4(P["emb_g2m"]))
    m2g = _mlp_rows(m2g_e, P["emb_m2g"], w1=w1p4(P["emb_m2g"]))

    w1 = P["enc_edge"]["W1"]
    ts = _mm(grid, w1[H:2 * H])
    td = _mm(mesh, w1[2 * H:])
    ga, gb = _sc_gather2(ts, td, g2m_src_g, g2m_dst_g)
    g2m = _edge_update(g2m, ga, gb, w1[:H], P["enc_edge"])
    parts = _sc_scatter_mesh(g2m, g2m_dst_s, zeros_rows)
    mesh = _node_update(mesh, parts[0], parts[1], P["enc_node"])
    grid = _mlp_rows(grid, P["enc_grid"], res=True)

    for pe, pn in zip(P["proc_edge"], P["proc_node"]):
        w1 = pe["W1"]
        ts = _mm(mesh, w1[H:2 * H])
        td = _mm(mesh, w1[2 * H:])
        ga, gb = _sc_gather2(ts, td, m2m_src_g, m2m_dst_g)
        m2m = _edge_update(m2m, ga, gb, w1[:H], pe)
        parts = _sc_scatter_mesh(m2m, m2m_dst_s, zeros_rows)
        mesh = _node_update(mesh, parts[0], parts[1], pn)

    w1 = P["dec_edge"]["W1"]
    ts = _mm(mesh, w1[H:2 * H])
    td = _mm(grid, w1[2 * H:])
    ga, gb = _sc_gather2(ts, td, m2g_src_g, m2g_dst_g)
    m2g = _edge_update(m2g, ga, gb, w1[:H], P["dec_edge"])
    agg = _sc_scatter_grid(m2g, m2g_dst_s, zeros_rows)
    grid = _node_update(grid, agg, None, P["dec_node"])

    out = _mlp_rows(grid, P["final"], norm=False)
    return out[:NG]
